# baseline (speedup 1.0000x reference)
import jax
import jax.numpy as jnp
from jax.experimental import pallas as pl
from jax.experimental.pallas import tpu as pltpu

_N_USERS = 2500
_N_ITEMS = 1400
_N_GROUPS = 196
_NTOT = _N_USERS + _N_ITEMS + _N_GROUPS
_HID = 64
_KITER = 10
_ALPHA = 0.1
_BM = 512
_NB = _NTOT // _BM


def _h_body(x_ref, wu_ref, bu_ref, wi_ref, bi_ref, wg_ref, bg_ref, h_ref):
    x = x_ref[...]
    r = jax.lax.broadcasted_iota(jnp.int32, (_NTOT, 1), 0)
    hu = jnp.maximum(jnp.dot(x, wu_ref[...], preferred_element_type=jnp.float32)
                     + bu_ref[...], 0.0)
    hi = jnp.maximum(jnp.dot(x, wi_ref[...], preferred_element_type=jnp.float32)
                     + bi_ref[...], 0.0)
    hg = jnp.maximum(jnp.dot(x, wg_ref[...], preferred_element_type=jnp.float32)
                     + bg_ref[...], 0.0)
    h_ref[...] = jnp.where(r < _N_USERS, hu,
                           jnp.where(r < _N_USERS + _N_ITEMS, hi, hg))


_BM_A = 64
_NB_A = _NTOT // _BM_A
_NRING = 6
_BM_C = 2048
_NB_C = _NTOT // _BM_C
_PH_A = _NB_A
_PH_B = _PH_A + (_KITER - 1) * _NB_C
_GRID = _PH_B + _KITER * _NB_C

_FP8 = jnp.float8_e4m3fn


def _prop_step(a_ref, zs_ref, h_ref, wh_ref, it, jj, bm):
    p = jax.lax.rem(it, 2)
    a = a_ref[pl.ds(jj * bm, bm), :]
    az = jnp.dot(a, zs_ref[p], preferred_element_type=jnp.float32)
    azw = jnp.dot(az.astype(jnp.bfloat16), wh_ref[...],
                  preferred_element_type=jnp.float32)
    hblk = h_ref[pl.ds(jj * bm, bm), :]
    znew = (1.0 - _ALPHA) * jnp.maximum(azw, 0.0) + _ALPHA * hblk
    zs_ref[1 - p, pl.ds(jj * bm, bm), :] = znew.astype(_FP8)
    return znew


def _blk_copy(hbm_ref, ring_ref, sem_ref, b):
    s = jax.lax.rem(b, _NRING)
    return pltpu.make_async_copy(
        hbm_ref.at[pl.ds(b * _BM_A, _BM_A), :], ring_ref.at[s], sem_ref.at[s])


def _prop_body(agu_ref, aui_ref, h_ref, wh_ref,
               ogu_ref, oui_ref, agu8_ref, aui8_ref, zsg_ref, zsu_ref,
               gring_ref, uring_ref, gsem_ref, usem_ref):
    i = pl.program_id(0)

    @pl.when(i == 0)
    def _init():
        h8 = h_ref[...].astype(_FP8)
        zsg_ref[0] = h8
        zsu_ref[0] = h8

    @pl.when(i == 0)
    def _prologue():
        for b in range(_NRING):
            _blk_copy(agu_ref, gring_ref, gsem_ref, b).start()

    @pl.when(i < _PH_A)
    def _phase_a():
        _blk_copy(agu_ref, gring_ref, gsem_ref, i).wait()
        t = (gring_ref[jax.lax.rem(i, _NRING)] * float(_NTOT)).astype(_FP8)
        agu8_ref[pl.ds(i * _BM_A, _BM_A), :] = t
        az = jnp.dot(t, zsg_ref[0], preferred_element_type=jnp.float32)
        azw = jnp.dot(az.astype(jnp.bfloat16), wh_ref[...],
                      preferred_element_type=jnp.float32)
        hblk = h_ref[pl.ds(i * _BM_A, _BM_A), :]
        znew = (1.0 - _ALPHA) * jnp.maximum(azw, 0.0) + _ALPHA * hblk
        zsg_ref[1, pl.ds(i * _BM_A, _BM_A), :] = znew.astype(_FP8)

        @pl.when(i + _NRING < _NB_A)
        def _next():
            _blk_copy(agu_ref, gring_ref, gsem_ref, i + _NRING).start()

        @pl.when(i == _PH_A - 1)
        def _ui_prologue():
            for b in range(_NRING):
                _blk_copy(aui_ref, uring_ref, usem_ref, b).start()

    @pl.when((i >= _PH_A) & (i < _PH_B))
    def _phase_b():
        q = i - _PH_A

        @pl.when(q < _NB_A // 4)
        def _cache_ui():
            for half in range(4):
                b = 4 * q + half
                _blk_copy(aui_ref, uring_ref, usem_ref, b).wait()
                aui8_ref[pl.ds(b * _BM_A, _BM_A), :] = (
                    uring_ref[jax.lax.rem(b, _NRING)]
                    * float(_NTOT)).astype(_FP8)

                @pl.when(b + _NRING < _NB_A)
                def _next():
                    _blk_copy(aui_ref, uring_ref, usem_ref, b + _NRING).start()

        it = q // _NB_C + 1
        jj = q % _NB_C
        znew = _prop_step(agu8_ref, zsg_ref, h_ref, wh_ref, it, jj, _BM_C)

        @pl.when(it == _KITER - 1)
        def _emit():
            ogu_ref[...] = znew

    @pl.when(i >= _PH_B)
    def _phase_c():
        r = i - _PH_B
        it = r // _NB_C
        jj = r % _NB_C
        znew = _prop_step(aui8_ref, zsu_ref, h_ref, wh_ref, it, jj, _BM_C)

        @pl.when(it == _KITER - 1)
        def _emit():
            oui_ref[...] = znew


def _propagate2(adj_gu, adj_ui, h, wh16):
    return pl.pallas_call(
        _prop_body,
        grid=(_GRID,),
        in_specs=[
            pl.BlockSpec(memory_space=pltpu.HBM),
            pl.BlockSpec(memory_space=pltpu.HBM),
            pl.BlockSpec((_NTOT, _HID), lambda i: (0, 0)),
            pl.BlockSpec((_HID, _HID), lambda i: (0, 0)),
        ],
        out_specs=[
            pl.BlockSpec((_BM_C, _HID),
                         lambda i: (jnp.clip(i - (_PH_B - _NB_C),
                                             0, _NB_C - 1), 0)),
            pl.BlockSpec((_BM_C, _HID),
                         lambda i: (jnp.clip(i - (_GRID - _NB_C),
                                             0, _NB_C - 1), 0)),
        ],
        out_shape=[
            jax.ShapeDtypeStruct((_NTOT, _HID), jnp.float32),
            jax.ShapeDtypeStruct((_NTOT, _HID), jnp.float32),
        ],
        scratch_shapes=[
            pltpu.VMEM((_NTOT, _NTOT), _FP8),
            pltpu.VMEM((_NTOT, _NTOT), _FP8),
            pltpu.VMEM((2, _NTOT, _HID), _FP8),
            pltpu.VMEM((2, _NTOT, _HID), _FP8),
            pltpu.VMEM((_NRING, _BM_A, _NTOT), jnp.float32),
            pltpu.VMEM((_NRING, _BM_A, _NTOT), jnp.float32),
            pltpu.SemaphoreType.DMA((_NRING,)),
            pltpu.SemaphoreType.DMA((_NRING,)),
        ],
    )(adj_gu, adj_ui, h, wh16)


def _merge_body(zg_ref, zu_ref, wq_ref, bq_ref, wk_ref, bk_ref,
                wv_ref, bv_ref, wfm_ref, bfm_ref, out_ref):
    zg = zg_ref[...]
    zu = zu_ref[...]
    wq = wq_ref[...]
    wk = wk_ref[...]
    wv = wv_ref[...]
    qg = jnp.dot(zg, wq, preferred_element_type=jnp.float32) + bq_ref[...]
    qu = jnp.dot(zu, wq, preferred_element_type=jnp.float32) + bq_ref[...]
    kg = jnp.dot(zg, wk, preferred_element_type=jnp.float32) + bk_ref[...]
    ku = jnp.dot(zu, wk, preferred_element_type=jnp.float32) + bk_ref[...]
    vg = jnp.dot(zg, wv, preferred_element_type=jnp.float32) + bv_ref[...]
    vu = jnp.dot(zu, wv, preferred_element_type=jnp.float32) + bv_ref[...]
    inv = 1.0 / (_HID ** 0.5)
    s00 = jnp.sum(qg * kg, axis=1, keepdims=True) * inv
    s01 = jnp.sum(qg * ku, axis=1, keepdims=True) * inv
    s10 = jnp.sum(qu * kg, axis=1, keepdims=True) * inv
    s11 = jnp.sum(qu * ku, axis=1, keepdims=True) * inv
    m0 = jnp.maximum(s00, s01)
    e00 = jnp.exp(s00 - m0)
    e01 = jnp.exp(s01 - m0)
    d0 = e00 + e01
    m1 = jnp.maximum(s10, s11)
    e10 = jnp.exp(s10 - m1)
    e11 = jnp.exp(s11 - m1)
    d1 = e10 + e11
    y0 = (e00 / d0) * vg + (e01 / d0) * vu
    y1 = (e10 / d1) * vg + (e11 / d1) * vu
    y = jnp.concatenate([y0, y1], axis=1)
    out_ref[...] = (jnp.dot(y, wfm_ref[...], preferred_element_type=jnp.float32)
                    + bfm_ref[...])


def kernel(x, adj_gu, adj_ui, Wu, bu, Wi, bi, Wg, bg, W_h,
           Wq, bq, Wk, bk, Wv, bv, Wfm, bfm):
    h = pl.pallas_call(
        _h_body,
        out_shape=jax.ShapeDtypeStruct((_NTOT, _HID), jnp.float32),
    )(x, Wu.T, bu.reshape(1, _HID), Wi.T, bi.reshape(1, _HID),
      Wg.T, bg.reshape(1, _HID))

    wh16 = (W_h / float(_NTOT)).astype(jnp.bfloat16)
    z_gu, z_ui = _propagate2(adj_gu, adj_ui, h, wh16)

    z_final = pl.pallas_call(
        _merge_body,
        out_shape=jax.ShapeDtypeStruct((_NTOT, _HID), jnp.float32),
    )(z_gu, z_ui, Wq.T, bq.reshape(1, _HID), Wk.T, bk.reshape(1, _HID),
      Wv.T, bv.reshape(1, _HID // 2), Wfm.T, bfm.reshape(1, _HID))

    return z_final, h


# speedup vs baseline: 1.0523x; 1.0523x over previous
import jax
import jax.numpy as jnp
from jax.experimental import pallas as pl
from jax.experimental.pallas import tpu as pltpu

_N_USERS = 2500
_N_ITEMS = 1400
_N_GROUPS = 196
_NTOT = _N_USERS + _N_ITEMS + _N_GROUPS
_HID = 64
_KITER = 10
_ALPHA = 0.1
_BM = 512
_NB = _NTOT // _BM


def _h_body(x_ref, wu_ref, bu_ref, wi_ref, bi_ref, wg_ref, bg_ref, h_ref):
    x = x_ref[...]
    r = jax.lax.broadcasted_iota(jnp.int32, (_NTOT, 1), 0)
    hu = jnp.maximum(jnp.dot(x, wu_ref[...], preferred_element_type=jnp.float32)
                     + bu_ref[...], 0.0)
    hi = jnp.maximum(jnp.dot(x, wi_ref[...], preferred_element_type=jnp.float32)
                     + bi_ref[...], 0.0)
    hg = jnp.maximum(jnp.dot(x, wg_ref[...], preferred_element_type=jnp.float32)
                     + bg_ref[...], 0.0)
    h_ref[...] = jnp.where(r < _N_USERS, hu,
                           jnp.where(r < _N_USERS + _N_ITEMS, hi, hg))


_BM_A = 128
_NB_A = _NTOT // _BM_A
_NRING = 3
_BM_C = 2048
_NB_C = _NTOT // _BM_C
_PH_A = _NB_A
_PH_B = _PH_A + (_KITER - 1) * _NB_C
_GRID = _PH_B + _KITER * _NB_C

_FP8 = jnp.float8_e4m3fn


def _prop_step(a_ref, zs_ref, h_ref, wh_ref, it, jj, bm):
    p = jax.lax.rem(it, 2)
    a = a_ref[pl.ds(jj * bm, bm), :]
    az = jnp.dot(a, zs_ref[p], preferred_element_type=jnp.float32)
    azw = jnp.dot(az.astype(jnp.bfloat16), wh_ref[...],
                  preferred_element_type=jnp.float32)
    hblk = h_ref[pl.ds(jj * bm, bm), :]
    znew = (1.0 - _ALPHA) * jnp.maximum(azw, 0.0) + _ALPHA * hblk
    zs_ref[1 - p, pl.ds(jj * bm, bm), :] = znew.astype(_FP8)
    return znew


def _blk_copy(hbm_ref, ring_ref, sem_ref, b):
    s = jax.lax.rem(b, _NRING)
    return pltpu.make_async_copy(
        hbm_ref.at[pl.ds(b * _BM_A, _BM_A), :], ring_ref.at[s], sem_ref.at[s])


def _prop_body(agu_ref, aui_ref, h_ref, wh_ref,
               ogu_ref, oui_ref, agu8_ref, aui8_ref, zsg_ref, zsu_ref,
               gring_ref, uring_ref, gsem_ref, usem_ref):
    i = pl.program_id(0)

    @pl.when(i == 0)
    def _init():
        h8 = h_ref[...].astype(_FP8)
        zsg_ref[0] = h8
        zsu_ref[0] = h8

    @pl.when(i == 0)
    def _prologue():
        for b in range(_NRING):
            _blk_copy(agu_ref, gring_ref, gsem_ref, b).start()

    @pl.when(i < _PH_A)
    def _phase_a():
        _blk_copy(agu_ref, gring_ref, gsem_ref, i).wait()
        t = (gring_ref[jax.lax.rem(i, _NRING)] * float(_NTOT)).astype(_FP8)
        agu8_ref[pl.ds(i * _BM_A, _BM_A), :] = t
        az = jnp.dot(t, zsg_ref[0], preferred_element_type=jnp.float32)
        azw = jnp.dot(az.astype(jnp.bfloat16), wh_ref[...],
                      preferred_element_type=jnp.float32)
        hblk = h_ref[pl.ds(i * _BM_A, _BM_A), :]
        znew = (1.0 - _ALPHA) * jnp.maximum(azw, 0.0) + _ALPHA * hblk
        zsg_ref[1, pl.ds(i * _BM_A, _BM_A), :] = znew.astype(_FP8)

        @pl.when(i + _NRING < _NB_A)
        def _next():
            _blk_copy(agu_ref, gring_ref, gsem_ref, i + _NRING).start()

        @pl.when(i == _PH_A - 1)
        def _ui_prologue():
            for b in range(_NRING):
                _blk_copy(aui_ref, uring_ref, usem_ref, b).start()

    @pl.when((i >= _PH_A) & (i < _PH_B))
    def _phase_b():
        q = i - _PH_A

        @pl.when(q < _NB_A // 2)
        def _cache_ui():
            for half in range(2):
                b = 2 * q + half
                _blk_copy(aui_ref, uring_ref, usem_ref, b).wait()
                aui8_ref[pl.ds(b * _BM_A, _BM_A), :] = (
                    uring_ref[jax.lax.rem(b, _NRING)]
                    * float(_NTOT)).astype(_FP8)

                @pl.when(b + _NRING < _NB_A)
                def _next():
                    _blk_copy(aui_ref, uring_ref, usem_ref, b + _NRING).start()

        it = q // _NB_C + 1
        jj = q % _NB_C
        znew = _prop_step(agu8_ref, zsg_ref, h_ref, wh_ref, it, jj, _BM_C)

        @pl.when(it == _KITER - 1)
        def _emit():
            ogu_ref[...] = znew

    @pl.when(i >= _PH_B)
    def _phase_c():
        r = i - _PH_B
        it = r // _NB_C
        jj = r % _NB_C
        znew = _prop_step(aui8_ref, zsu_ref, h_ref, wh_ref, it, jj, _BM_C)

        @pl.when(it == _KITER - 1)
        def _emit():
            oui_ref[...] = znew


def _propagate2(adj_gu, adj_ui, h, wh16):
    return pl.pallas_call(
        _prop_body,
        grid=(_GRID,),
        in_specs=[
            pl.BlockSpec(memory_space=pltpu.HBM),
            pl.BlockSpec(memory_space=pltpu.HBM),
            pl.BlockSpec((_NTOT, _HID), lambda i: (0, 0)),
            pl.BlockSpec((_HID, _HID), lambda i: (0, 0)),
        ],
        out_specs=[
            pl.BlockSpec((_BM_C, _HID),
                         lambda i: (jnp.clip(i - (_PH_B - _NB_C),
                                             0, _NB_C - 1), 0)),
            pl.BlockSpec((_BM_C, _HID),
                         lambda i: (jnp.clip(i - (_GRID - _NB_C),
                                             0, _NB_C - 1), 0)),
        ],
        out_shape=[
            jax.ShapeDtypeStruct((_NTOT, _HID), jnp.float32),
            jax.ShapeDtypeStruct((_NTOT, _HID), jnp.float32),
        ],
        scratch_shapes=[
            pltpu.VMEM((_NTOT, _NTOT), _FP8),
            pltpu.VMEM((_NTOT, _NTOT), _FP8),
            pltpu.VMEM((2, _NTOT, _HID), _FP8),
            pltpu.VMEM((2, _NTOT, _HID), _FP8),
            pltpu.VMEM((_NRING, _BM_A, _NTOT), jnp.float32),
            pltpu.VMEM((_NRING, _BM_A, _NTOT), jnp.float32),
            pltpu.SemaphoreType.DMA((_NRING,)),
            pltpu.SemaphoreType.DMA((_NRING,)),
        ],
    )(adj_gu, adj_ui, h, wh16)


def _merge_body(zg_ref, zu_ref, wq_ref, bq_ref, wk_ref, bk_ref,
                wv_ref, bv_ref, wfm_ref, bfm_ref, out_ref):
    zg = zg_ref[...]
    zu = zu_ref[...]
    wq = wq_ref[...]
    wk = wk_ref[...]
    wv = wv_ref[...]
    qg = jnp.dot(zg, wq, preferred_element_type=jnp.float32) + bq_ref[...]
    qu = jnp.dot(zu, wq, preferred_element_type=jnp.float32) + bq_ref[...]
    kg = jnp.dot(zg, wk, preferred_element_type=jnp.float32) + bk_ref[...]
    ku = jnp.dot(zu, wk, preferred_element_type=jnp.float32) + bk_ref[...]
    vg = jnp.dot(zg, wv, preferred_element_type=jnp.float32) + bv_ref[...]
    vu = jnp.dot(zu, wv, preferred_element_type=jnp.float32) + bv_ref[...]
    inv = 1.0 / (_HID ** 0.5)
    s00 = jnp.sum(qg * kg, axis=1, keepdims=True) * inv
    s01 = jnp.sum(qg * ku, axis=1, keepdims=True) * inv
    s10 = jnp.sum(qu * kg, axis=1, keepdims=True) * inv
    s11 = jnp.sum(qu * ku, axis=1, keepdims=True) * inv
    m0 = jnp.maximum(s00, s01)
    e00 = jnp.exp(s00 - m0)
    e01 = jnp.exp(s01 - m0)
    d0 = e00 + e01
    m1 = jnp.maximum(s10, s11)
    e10 = jnp.exp(s10 - m1)
    e11 = jnp.exp(s11 - m1)
    d1 = e10 + e11
    y0 = (e00 / d0) * vg + (e01 / d0) * vu
    y1 = (e10 / d1) * vg + (e11 / d1) * vu
    y = jnp.concatenate([y0, y1], axis=1)
    out_ref[...] = (jnp.dot(y, wfm_ref[...], preferred_element_type=jnp.float32)
                    + bfm_ref[...])


def kernel(x, adj_gu, adj_ui, Wu, bu, Wi, bi, Wg, bg, W_h,
           Wq, bq, Wk, bk, Wv, bv, Wfm, bfm):
    h = pl.pallas_call(
        _h_body,
        out_shape=jax.ShapeDtypeStruct((_NTOT, _HID), jnp.float32),
    )(x, Wu.T, bu.reshape(1, _HID), Wi.T, bi.reshape(1, _HID),
      Wg.T, bg.reshape(1, _HID))

    wh16 = (W_h / float(_NTOT)).astype(jnp.bfloat16)
    z_gu, z_ui = _propagate2(adj_gu, adj_ui, h, wh16)

    z_final = pl.pallas_call(
        _merge_body,
        out_shape=jax.ShapeDtypeStruct((_NTOT, _HID), jnp.float32),
    )(z_gu, z_ui, Wq.T, bq.reshape(1, _HID), Wk.T, bk.reshape(1, _HID),
      Wv.T, bv.reshape(1, _HID // 2), Wfm.T, bfm.reshape(1, _HID))

    return z_final, h
